# Initial kernel scaffold; baseline (speedup 1.0000x reference)
import jax
import jax.numpy as jnp
from jax.experimental import pallas as pl


def kernel(x, edge_index, weights, length, W_enc, b_enc, W_m1, W_m2, b_m, W_upd, b_upd):
    raise NotImplementedError("write your pallas kernel here")


# speedup vs baseline: 2.1953x; 2.1953x over previous
import functools

import jax
import jax.numpy as jnp
from jax import lax
from jax.experimental import pallas as pl
from jax.experimental.pallas import tpu as pltpu
from jax.experimental.pallas import tpu_sc as plsc

N = 10000
H = 128
MSG_STEPS = 3

_NC = 2
_NS = 16
_NW = _NC * _NS
_L = 16
_C = 128
_ROW_BLK = 2000
_NP = 10240


def _enc_body(x_ref, w_ref, b_ref, o_ref):
    o_ref[...] = jnp.maximum(
        jnp.dot(x_ref[...], w_ref[...], preferred_element_type=jnp.float32)
        + b_ref[...], 0.0)


def _transform_body(ih_ref, h_ref, w1_ref, w2_ref, bm_ref, ts_ref, td_ref):
    z = jnp.concatenate([ih_ref[...], h_ref[...]], axis=1)
    ts_ref[...] = jnp.dot(z, w1_ref[...], preferred_element_type=jnp.float32)
    td_ref[...] = (
        jnp.dot(z, w2_ref[...], preferred_element_type=jnp.float32)
        + bm_ref[...])


def _update_body(ih_ref, h_ref, a0_ref, a1_ref, wu_ref, bu_ref, o_ref):
    agg = a0_ref[...] + a1_ref[...]
    z3 = jnp.concatenate([ih_ref[...], h_ref[...], agg], axis=1)
    o_ref[...] = jnp.maximum(
        jnp.dot(z3, wu_ref[...], preferred_element_type=jnp.float32)
        + bu_ref[...], 0.0)


def _row_spec(cols):
    return pl.BlockSpec((_ROW_BLK, cols), lambda i: (i, 0))


def _full_spec(rows, cols):
    return pl.BlockSpec((rows, cols), lambda i: (0, 0))


_GRID = (N // _ROW_BLK,)
_F32 = jnp.float32


def _encoder(x, w, b):
    return pl.pallas_call(
        _enc_body,
        grid=_GRID,
        in_specs=[_row_spec(H), _full_spec(H, H), _full_spec(1, H)],
        out_specs=_row_spec(H),
        out_shape=jax.ShapeDtypeStruct((N, H), _F32),
    )(x, w, b)


def _transform(ih, h, w1, w2, bm):
    return pl.pallas_call(
        _transform_body,
        grid=_GRID,
        in_specs=[_row_spec(H), _row_spec(H), _full_spec(2 * H, H),
                  _full_spec(2 * H, H), _full_spec(1, H)],
        out_specs=[_row_spec(H), _row_spec(H)],
        out_shape=[jax.ShapeDtypeStruct((N, H), _F32),
                   jax.ShapeDtypeStruct((N, H), _F32)],
    )(ih, h, w1, w2, bm)


def _update(ih, h, agg2, wu, bu):
    return pl.pallas_call(
        _update_body,
        grid=_GRID,
        in_specs=[_row_spec(H), _row_spec(H), _row_spec(H), _row_spec(H),
                  _full_spec(3 * H, H), _full_spec(1, H)],
        out_specs=_row_spec(H),
        out_shape=jax.ShapeDtypeStruct((N, H), _F32),
    )(ih, h, agg2[0], agg2[1], wu, bu)


_SC8 = 8


def _edge_body(ts_hbm, td_hbm, src_hbm, dst_hbm, w_hbm, zeros_hbm, out_hbm,
               src_v, dst_v, w_v, rows_a, rows_b, agg_sh, sem_a, sem_b):
    c = lax.axis_index("c")
    s = lax.axis_index("s")
    wid = c * _NS + s
    ch = src_hbm.shape[0] // _NW
    rpt = agg_sh.shape[0] // _NS

    pltpu.sync_copy(zeros_hbm.at[pl.ds(s * rpt, rpt)],
                    agg_sh.at[pl.ds(s * rpt, rpt)])
    plsc.subcore_barrier()

    def super_body(sc, carry):
        base = wid * ch + sc * _SC8
        pltpu.sync_copy(src_hbm.at[pl.ds(base, _SC8)], src_v)
        pltpu.sync_copy(dst_hbm.at[pl.ds(base, _SC8)], dst_v)
        pltpu.sync_copy(w_hbm.at[pl.ds(base, _SC8)], w_v)
        for j in range(_SC8):
            ga = pltpu.async_copy(ts_hbm.at[src_v.at[j]], rows_a, sem_a)
            gb = pltpu.async_copy(td_hbm.at[dst_v.at[j]], rows_b, sem_b)
            ga.wait()
            gb.wait()

            for g in range(_C // _L):
                wg = w_v[j, pl.ds(g * _L, _L)]

                def edge_body(e16, carry2, g=g, wg=wg):
                    wv = lax.gather(
                        wg, jnp.full((_L, 1), e16, jnp.int32),
                        lax.GatherDimensionNumbers(
                            offset_dims=(), collapsed_slice_dims=(0,),
                            start_index_map=(0,)),
                        slice_sizes=(1,),
                        mode=lax.GatherScatterMode.PROMISE_IN_BOUNDS)
                    e = g * _L + e16
                    for hh in range(H // _L):
                        sl = pl.ds(hh * _L, _L)
                        m = jnp.maximum(rows_a[e, sl] + rows_b[e, sl],
                                        0.0) * wv
                        rows_a[e, sl] = m
                    return carry2

                lax.fori_loop(0, _L, edge_body, 0)
            pltpu.sync_copy(rows_a, agg_sh.at[dst_v.at[j]], add=True)
        return carry

    lax.fori_loop(0, ch // _SC8, super_body, 0)
    plsc.subcore_barrier()
    pltpu.sync_copy(agg_sh.at[pl.ds(s * rpt, rpt)],
                    out_hbm.at[c, pl.ds(s * rpt, rpt)])


def _edge_stage(ts, td, src2, dst2, w2, zeros):
    ch_total = src2.shape[0]
    mesh = plsc.VectorSubcoreMesh(core_axis_name="c", subcore_axis_name="s")
    fn = pl.kernel(
        _edge_body,
        out_type=jax.ShapeDtypeStruct((_NC, _NP, H), _F32),
        mesh=mesh,
        scratch_types=[
            pltpu.VMEM((_SC8, _C), jnp.int32),
            pltpu.VMEM((_SC8, _C), jnp.int32),
            pltpu.VMEM((_SC8, _C), _F32),
            pltpu.VMEM((_C, H), _F32),
            pltpu.VMEM((_C, H), _F32),
            pltpu.VMEM_SHARED((_NP, H), _F32),
            pltpu.SemaphoreType.DMA,
            pltpu.SemaphoreType.DMA,
        ],
    )
    return fn(ts, td, src2, dst2, w2, zeros)


def kernel(x, edge_index, weights, length, W_enc, b_enc, W_m1, W_m2, b_m,
           W_upd, b_upd):
    E = weights.shape[0]
    ch_tile = ((-(-E // (_NW * _C)) + 7) // 8) * 8
    ch_total = ch_tile * _NW
    e_pad = ch_total * _C

    src = edge_index[0]
    dst = edge_index[1]
    pad = e_pad - E
    src2 = jnp.concatenate([src, jnp.zeros((pad,), jnp.int32)]).reshape(
        ch_total, _C)
    dst2 = jnp.concatenate([dst, jnp.zeros((pad,), jnp.int32)]).reshape(
        ch_total, _C)
    w2 = jnp.concatenate([weights, jnp.zeros((pad,), _F32)]).reshape(
        ch_total, _C)
    zeros = jnp.zeros((_NP, H), _F32)

    b_enc2 = b_enc.reshape(1, H)
    b_m2 = b_m.reshape(1, H)
    b_upd2 = b_upd.reshape(1, H)

    ih = _encoder(x, W_enc, b_enc2)

    def step_body(_step, hidden):
        for _ in range(MSG_STEPS):
            ts, td = _transform(ih, hidden, W_m1, W_m2, b_m2)
            agg2 = _edge_stage(ts, td, src2, dst2, w2, zeros)
            hidden = _update(ih, hidden, agg2, W_upd, b_upd2)
        return hidden

    hidden = lax.fori_loop(0, length, step_body, ih)
    return jnp.concatenate([ih, hidden], axis=-1)


# speedup vs baseline: 3.3647x; 1.5326x over previous
import functools

import jax
import jax.numpy as jnp
from jax import lax
from jax.experimental import pallas as pl
from jax.experimental.pallas import tpu as pltpu
from jax.experimental.pallas import tpu_sc as plsc

N = 10000
H = 128
MSG_STEPS = 3

_NC = 2
_NS = 16
_NW = _NC * _NS
_L = 16
_C = 128
_ROW_BLK = 2000
_NP = 10240


def _enc_body(x_ref, w_ref, b_ref, o_ref):
    o_ref[...] = jnp.maximum(
        jnp.dot(x_ref[...], w_ref[...], preferred_element_type=jnp.float32)
        + b_ref[...], 0.0)


def _transform_body(ih_ref, h_ref, w1_ref, w2_ref, bm_ref, ts_ref, td_ref):
    z = jnp.concatenate([ih_ref[...], h_ref[...]], axis=1)
    ts_ref[...] = jnp.dot(z, w1_ref[...], preferred_element_type=jnp.float32)
    td_ref[...] = (
        jnp.dot(z, w2_ref[...], preferred_element_type=jnp.float32)
        + bm_ref[...])


def _update_body(ih_ref, h_ref, a0_ref, a1_ref, wu_ref, bu_ref, o_ref):
    agg = a0_ref[...] + a1_ref[...]
    z3 = jnp.concatenate([ih_ref[...], h_ref[...], agg], axis=1)
    o_ref[...] = jnp.maximum(
        jnp.dot(z3, wu_ref[...], preferred_element_type=jnp.float32)
        + bu_ref[...], 0.0)


def _row_spec(cols):
    return pl.BlockSpec((_ROW_BLK, cols), lambda i: (i, 0))


def _full_spec(rows, cols):
    return pl.BlockSpec((rows, cols), lambda i: (0, 0))


_GRID = (N // _ROW_BLK,)
_F32 = jnp.float32


def _encoder(x, w, b):
    return pl.pallas_call(
        _enc_body,
        grid=_GRID,
        in_specs=[_row_spec(H), _full_spec(H, H), _full_spec(1, H)],
        out_specs=_row_spec(H),
        out_shape=jax.ShapeDtypeStruct((N, H), _F32),
    )(x, w, b)


def _transform(ih, h, w1, w2, bm):
    return pl.pallas_call(
        _transform_body,
        grid=_GRID,
        in_specs=[_row_spec(H), _row_spec(H), _full_spec(2 * H, H),
                  _full_spec(2 * H, H), _full_spec(1, H)],
        out_specs=[_row_spec(H), _row_spec(H)],
        out_shape=[jax.ShapeDtypeStruct((N, H), _F32),
                   jax.ShapeDtypeStruct((N, H), _F32)],
    )(ih, h, w1, w2, bm)


def _update(ih, h, agg2, wu, bu):
    return pl.pallas_call(
        _update_body,
        grid=_GRID,
        in_specs=[_row_spec(H), _row_spec(H), _row_spec(H), _row_spec(H),
                  _full_spec(3 * H, H), _full_spec(1, H)],
        out_specs=_row_spec(H),
        out_shape=jax.ShapeDtypeStruct((N, H), _F32),
    )(ih, h, agg2[0], agg2[1], wu, bu)


_SC8 = 8
_CE = 64
_NCK = (_SC8 * _C) // _CE


def _edge_body(ts_hbm, td_hbm, src_hbm, dst_hbm, w_hbm, zeros_hbm, out_hbm,
               src_v, dst_v, w_v, ra0, rb0, ra1, rb1, agg_sh,
               sem_a0, sem_b0, sem_a1, sem_b1):
    c = lax.axis_index("c")
    s = lax.axis_index("s")
    wid = c * _NS + s
    ch = src_hbm.shape[0] // _NW
    rpt = agg_sh.shape[0] // _NS
    bufs = ((ra0, rb0, sem_a0, sem_b0), (ra1, rb1, sem_a1, sem_b1))

    def _idx(j):
        return (j // 2, pl.ds((j % 2) * _CE, _CE))

    def _gather(j, parity):
        ra, rb, sa, sb = bufs[parity]
        pltpu.async_copy(ts_hbm.at[src_v.at[_idx(j)]], ra, sa)
        pltpu.async_copy(td_hbm.at[dst_v.at[_idx(j)]], rb, sb)

    pltpu.sync_copy(zeros_hbm.at[pl.ds(s * rpt, rpt)],
                    agg_sh.at[pl.ds(s * rpt, rpt)])
    plsc.subcore_barrier()

    def super_body(sc, carry):
        base = wid * ch + sc * _SC8
        pltpu.sync_copy(src_hbm.at[pl.ds(base, _SC8)], src_v)
        pltpu.sync_copy(dst_hbm.at[pl.ds(base, _SC8)], dst_v)
        pltpu.sync_copy(w_hbm.at[pl.ds(base, _SC8)], w_v)
        _gather(0, 0)
        for j in range(_NCK):
            cur = j % 2
            if j + 1 < _NCK:
                _gather(j + 1, 1 - cur)
            ra, rb, sa, sb = bufs[cur]
            pltpu.make_async_copy(ts_hbm.at[src_v.at[_idx(j)]], ra, sa).wait()
            pltpu.make_async_copy(td_hbm.at[dst_v.at[_idx(j)]], rb, sb).wait()

            jr, jc = j // 2, (j % 2) * _CE
            for g in range(_CE // _L):
                wg = w_v[jr, pl.ds(jc + g * _L, _L)]

                def edge_body(e16, carry2, g=g, wg=wg, ra=ra, rb=rb):
                    wv = lax.gather(
                        wg, jnp.full((_L, 1), e16, jnp.int32),
                        lax.GatherDimensionNumbers(
                            offset_dims=(), collapsed_slice_dims=(0,),
                            start_index_map=(0,)),
                        slice_sizes=(1,),
                        mode=lax.GatherScatterMode.PROMISE_IN_BOUNDS)
                    e = g * _L + e16
                    for hh in range(H // _L):
                        sl = pl.ds(hh * _L, _L)
                        m = jnp.maximum(ra[e, sl] + rb[e, sl], 0.0) * wv
                        ra[e, sl] = m
                    return carry2

                lax.fori_loop(0, _L, edge_body, 0)
            pltpu.sync_copy(ra, agg_sh.at[dst_v.at[_idx(j)]], add=True)
        return carry

    lax.fori_loop(0, ch // _SC8, super_body, 0)
    plsc.subcore_barrier()
    pltpu.sync_copy(agg_sh.at[pl.ds(s * rpt, rpt)],
                    out_hbm.at[c, pl.ds(s * rpt, rpt)])


def _edge_stage(ts, td, src2, dst2, w2, zeros):
    ch_total = src2.shape[0]
    mesh = plsc.VectorSubcoreMesh(core_axis_name="c", subcore_axis_name="s")
    fn = pl.kernel(
        _edge_body,
        out_type=jax.ShapeDtypeStruct((_NC, _NP, H), _F32),
        mesh=mesh,
        scratch_types=[
            pltpu.VMEM((_SC8, _C), jnp.int32),
            pltpu.VMEM((_SC8, _C), jnp.int32),
            pltpu.VMEM((_SC8, _C), _F32),
            pltpu.VMEM((_CE, H), _F32),
            pltpu.VMEM((_CE, H), _F32),
            pltpu.VMEM((_CE, H), _F32),
            pltpu.VMEM((_CE, H), _F32),
            pltpu.VMEM_SHARED((_NP, H), _F32),
            pltpu.SemaphoreType.DMA,
            pltpu.SemaphoreType.DMA,
            pltpu.SemaphoreType.DMA,
            pltpu.SemaphoreType.DMA,
        ],
    )
    return fn(ts, td, src2, dst2, w2, zeros)


def kernel(x, edge_index, weights, length, W_enc, b_enc, W_m1, W_m2, b_m,
           W_upd, b_upd):
    E = weights.shape[0]
    ch_tile = ((-(-E // (_NW * _C)) + 7) // 8) * 8
    ch_total = ch_tile * _NW
    e_pad = ch_total * _C

    src = edge_index[0]
    dst = edge_index[1]
    pad = e_pad - E
    src2 = jnp.concatenate([src, jnp.zeros((pad,), jnp.int32)]).reshape(
        ch_total, _C)
    dst2 = jnp.concatenate([dst, jnp.zeros((pad,), jnp.int32)]).reshape(
        ch_total, _C)
    w2 = jnp.concatenate([weights, jnp.zeros((pad,), _F32)]).reshape(
        ch_total, _C)
    zeros = jnp.zeros((_NP, H), _F32)

    b_enc2 = b_enc.reshape(1, H)
    b_m2 = b_m.reshape(1, H)
    b_upd2 = b_upd.reshape(1, H)

    ih = _encoder(x, W_enc, b_enc2)

    def step_body(_step, hidden):
        for _ in range(MSG_STEPS):
            ts, td = _transform(ih, hidden, W_m1, W_m2, b_m2)
            agg2 = _edge_stage(ts, td, src2, dst2, w2, zeros)
            hidden = _update(ih, hidden, agg2, W_upd, b_upd2)
        return hidden

    hidden = lax.fori_loop(0, length, step_body, ih)
    return jnp.concatenate([ih, hidden], axis=-1)


# speedup vs baseline: 4.1638x; 1.2375x over previous
import functools

import jax
import jax.numpy as jnp
from jax import lax
from jax.experimental import pallas as pl
from jax.experimental.pallas import tpu as pltpu
from jax.experimental.pallas import tpu_sc as plsc

N = 10000
H = 128
MSG_STEPS = 3

_NC = 2
_NS = 16
_NW = _NC * _NS
_L = 16
_C = 128
_ROW_BLK = 2000
_NP = 10240


def _enc_body(x_ref, w_ref, b_ref, o_ref):
    o_ref[...] = jnp.maximum(
        jnp.dot(x_ref[...], w_ref[...], preferred_element_type=jnp.float32)
        + b_ref[...], 0.0)


def _transform_body(ih_ref, h_ref, w1_ref, w2_ref, bm_ref, ts_ref, td_ref):
    z = jnp.concatenate([ih_ref[...], h_ref[...]], axis=1)
    ts_ref[...] = jnp.dot(z, w1_ref[...], preferred_element_type=jnp.float32)
    td_ref[...] = (
        jnp.dot(z, w2_ref[...], preferred_element_type=jnp.float32)
        + bm_ref[...])


def _update_body(ih_ref, h_ref, a0_ref, a1_ref, wu_ref, bu_ref, o_ref):
    agg = a0_ref[...] + a1_ref[...]
    z3 = jnp.concatenate([ih_ref[...], h_ref[...], agg], axis=1)
    o_ref[...] = jnp.maximum(
        jnp.dot(z3, wu_ref[...], preferred_element_type=jnp.float32)
        + bu_ref[...], 0.0)


def _row_spec(cols):
    return pl.BlockSpec((_ROW_BLK, cols), lambda i: (i, 0))


def _full_spec(rows, cols):
    return pl.BlockSpec((rows, cols), lambda i: (0, 0))


_GRID = (N // _ROW_BLK,)
_F32 = jnp.float32


def _encoder(x, w, b):
    return pl.pallas_call(
        _enc_body,
        grid=_GRID,
        in_specs=[_row_spec(H), _full_spec(H, H), _full_spec(1, H)],
        out_specs=_row_spec(H),
        out_shape=jax.ShapeDtypeStruct((N, H), _F32),
    )(x, w, b)


def _transform(ih, h, w1, w2, bm):
    return pl.pallas_call(
        _transform_body,
        grid=_GRID,
        in_specs=[_row_spec(H), _row_spec(H), _full_spec(2 * H, H),
                  _full_spec(2 * H, H), _full_spec(1, H)],
        out_specs=[_row_spec(H), _row_spec(H)],
        out_shape=[jax.ShapeDtypeStruct((N, H), _F32),
                   jax.ShapeDtypeStruct((N, H), _F32)],
    )(ih, h, w1, w2, bm)


def _update(ih, h, agg2, wu, bu):
    return pl.pallas_call(
        _update_body,
        grid=_GRID,
        in_specs=[_row_spec(H), _row_spec(H), _row_spec(H), _row_spec(H),
                  _full_spec(3 * H, H), _full_spec(1, H)],
        out_specs=_row_spec(H),
        out_shape=jax.ShapeDtypeStruct((N, H), _F32),
    )(ih, h, agg2[0], agg2[1], wu, bu)


_SC8 = 8
_CE = 64
_NCK = (_SC8 * _C) // _CE


def _edge_body(ts_hbm, td_hbm, src_hbm, dst_hbm, w_hbm, zeros_hbm, out_hbm,
               src_v, dst_v, w_v, ra0, rb0, ra1, rb1, agg_sh,
               sem_a0, sem_b0, sem_a1, sem_b1):
    c = lax.axis_index("c")
    s = lax.axis_index("s")
    wid = c * _NS + s
    ch = src_hbm.shape[0] // _NW
    rpt = agg_sh.shape[0] // _NS
    bufs = ((ra0, rb0, sem_a0, sem_b0), (ra1, rb1, sem_a1, sem_b1))

    def _gather(row, col, parity):
        ra, rb, sa, sb = bufs[parity]
        pltpu.async_copy(ts_hbm.at[src_v.at[row, pl.ds(col, _CE)]], ra, sa)
        pltpu.async_copy(td_hbm.at[dst_v.at[row, pl.ds(col, _CE)]], rb, sb)

    def _wait(row, col, parity):
        ra, rb, sa, sb = bufs[parity]
        pltpu.make_async_copy(
            ts_hbm.at[src_v.at[row, pl.ds(col, _CE)]], ra, sa).wait()
        pltpu.make_async_copy(
            td_hbm.at[dst_v.at[row, pl.ds(col, _CE)]], rb, sb).wait()

    def _compute(row, col, parity):
        ra, rb, _, _ = bufs[parity]
        for g in range(_CE // _L):
            wg = w_v[row, pl.ds(col + g * _L, _L)]

            def edge_body(e16, g=g, wg=wg, ra=ra, rb=rb):
                wv = lax.gather(
                    wg, jnp.full((_L, 1), e16, jnp.int32),
                    lax.GatherDimensionNumbers(
                        offset_dims=(), collapsed_slice_dims=(0,),
                        start_index_map=(0,)),
                    slice_sizes=(1,),
                    mode=lax.GatherScatterMode.PROMISE_IN_BOUNDS)
                e = g * _L + e16
                for hh in range(H // _L):
                    sl = pl.ds(hh * _L, _L)
                    m = jnp.maximum(ra[e, sl] + rb[e, sl], 0.0) * wv
                    ra[e, sl] = m

            plsc.parallel_loop(0, _L, unroll=4)(edge_body)

    def _scatter(row, col, parity):
        ra = bufs[parity][0]
        pltpu.sync_copy(ra, agg_sh.at[dst_v.at[row, pl.ds(col, _CE)]],
                        add=True)

    pltpu.sync_copy(zeros_hbm.at[pl.ds(s * rpt, rpt)],
                    agg_sh.at[pl.ds(s * rpt, rpt)])
    plsc.subcore_barrier()

    def super_body(sc, carry):
        base = wid * ch + sc * _SC8
        pltpu.sync_copy(src_hbm.at[pl.ds(base, _SC8)], src_v)
        pltpu.sync_copy(dst_hbm.at[pl.ds(base, _SC8)], dst_v)
        pltpu.sync_copy(w_hbm.at[pl.ds(base, _SC8)], w_v)
        _gather(0, 0, 0)

        def pair_body(p, carry2):
            _gather(p, _CE, 1)
            _wait(p, 0, 0)
            _compute(p, 0, 0)
            _scatter(p, 0, 0)

            @pl.when(p < _SC8 - 1)
            def _():
                _gather(p + 1, 0, 0)

            _wait(p, _CE, 1)
            _compute(p, _CE, 1)
            _scatter(p, _CE, 1)
            return carry2

        lax.fori_loop(0, _SC8, pair_body, 0)
        return carry

    lax.fori_loop(0, ch // _SC8, super_body, 0)
    plsc.subcore_barrier()
    pltpu.sync_copy(agg_sh.at[pl.ds(s * rpt, rpt)],
                    out_hbm.at[c, pl.ds(s * rpt, rpt)])


def _edge_stage(ts, td, src2, dst2, w2, zeros):
    ch_total = src2.shape[0]
    mesh = plsc.VectorSubcoreMesh(core_axis_name="c", subcore_axis_name="s")
    fn = pl.kernel(
        _edge_body,
        out_type=jax.ShapeDtypeStruct((_NC, _NP, H), _F32),
        mesh=mesh,
        scratch_types=[
            pltpu.VMEM((_SC8, _C), jnp.int32),
            pltpu.VMEM((_SC8, _C), jnp.int32),
            pltpu.VMEM((_SC8, _C), _F32),
            pltpu.VMEM((_CE, H), _F32),
            pltpu.VMEM((_CE, H), _F32),
            pltpu.VMEM((_CE, H), _F32),
            pltpu.VMEM((_CE, H), _F32),
            pltpu.VMEM_SHARED((_NP, H), _F32),
            pltpu.SemaphoreType.DMA,
            pltpu.SemaphoreType.DMA,
            pltpu.SemaphoreType.DMA,
            pltpu.SemaphoreType.DMA,
        ],
    )
    return fn(ts, td, src2, dst2, w2, zeros)


def kernel(x, edge_index, weights, length, W_enc, b_enc, W_m1, W_m2, b_m,
           W_upd, b_upd):
    E = weights.shape[0]
    ch_tile = ((-(-E // (_NW * _C)) + 7) // 8) * 8
    ch_total = ch_tile * _NW
    e_pad = ch_total * _C

    src = edge_index[0]
    dst = edge_index[1]
    pad = e_pad - E
    src2 = jnp.concatenate([src, jnp.zeros((pad,), jnp.int32)]).reshape(
        ch_total, _C)
    dst2 = jnp.concatenate([dst, jnp.zeros((pad,), jnp.int32)]).reshape(
        ch_total, _C)
    w2 = jnp.concatenate([weights, jnp.zeros((pad,), _F32)]).reshape(
        ch_total, _C)
    zeros = jnp.zeros((_NP, H), _F32)

    b_enc2 = b_enc.reshape(1, H)
    b_m2 = b_m.reshape(1, H)
    b_upd2 = b_upd.reshape(1, H)

    ih = _encoder(x, W_enc, b_enc2)

    def step_body(_step, hidden):
        for _ in range(MSG_STEPS):
            ts, td = _transform(ih, hidden, W_m1, W_m2, b_m2)
            agg2 = _edge_stage(ts, td, src2, dst2, w2, zeros)
            hidden = _update(ih, hidden, agg2, W_upd, b_upd2)
        return hidden

    hidden = lax.fori_loop(0, length, step_body, ih)
    return jnp.concatenate([ih, hidden], axis=-1)
